# baseline (speedup 1.0000x reference)
import functools

import jax
import jax.numpy as jnp
from jax.experimental import pallas as pl
from jax.experimental.pallas import tpu as pltpu


def _encoder_kernel(x_ref, w1_ref, b1_ref, w2_ref, b2_ref, o_ref, *, H, W, C):
    bf16 = jnp.bfloat16
    HW = H * W
    xT = x_ref[...]
    xs1 = jnp.concatenate([xT[:, 1:], xT[:, :1]], axis=1)
    xs2 = jnp.concatenate([xT[:, 2:], xT[:, :2]], axis=1)
    xp = jnp.concatenate(
        [xT.astype(bf16), xs1.astype(bf16), xs2.astype(bf16)], axis=0)

    z1 = jnp.dot(w1_ref[...], xp, preferred_element_type=jnp.float32)

    b1 = pltpu.repeat(b1_ref[...], HW // C, axis=1)
    za = z1[0:C, :]
    zb = z1[C:2 * C, :]
    zb = jnp.concatenate([zb[:, W:], zb[:, :W]], axis=1)
    zc = z1[2 * C:3 * C, :]
    zc = jnp.concatenate([zc[:, 2 * W:], zc[:, :2 * W]], axis=1)
    h = jnp.maximum(za + zb + zc + b1, 0.0)

    hs1 = jnp.concatenate([h[:, 1:], h[:, :1]], axis=1)
    hs2 = jnp.concatenate([h[:, 2:], h[:, :2]], axis=1)
    hp = jnp.concatenate(
        [h.astype(bf16), hs1.astype(bf16), hs2.astype(bf16)], axis=0)

    z2 = jnp.dot(w2_ref[...], hp, preferred_element_type=jnp.float32)

    b2 = pltpu.repeat(b2_ref[...], HW // C, axis=1)
    ya = z2[0:C, :]
    yb = jnp.concatenate([z2[C:2 * C, W:], z2[C:2 * C, :W]], axis=1)
    yc = jnp.concatenate(
        [z2[2 * C:3 * C, 2 * W:], z2[2 * C:3 * C, :2 * W]], axis=1)
    y = jnp.maximum(ya + yb + yc + b2, 0.0)
    res = jnp.concatenate([xs2[:, 2 * W:], xs2[:, :2 * W]], axis=1)
    y = y + res

    Wo = W - 4
    pieces = [y[:, i * W:i * W + Wo] for i in range(H - 4)]
    o_ref[...] = jnp.concatenate(pieces, axis=1)


def kernel(x, w1, b1, w2, b2):
    N, C, H, W = x.shape
    bf16 = jnp.bfloat16
    xf = x.reshape(N, C, H * W)
    w1c = jnp.transpose(w1, (2, 0, 3, 1)).reshape(3 * C, 3 * C).astype(bf16)
    w2c = jnp.transpose(w2, (2, 0, 3, 1)).reshape(3 * C, 3 * C).astype(bf16)
    b1k = jnp.tile(b1.reshape(C, 1), (1, C)).astype(jnp.float32)
    b2k = jnp.tile(b2.reshape(C, 1), (1, C)).astype(jnp.float32)

    body = functools.partial(_encoder_kernel, H=H, W=W, C=C)
    out = pl.pallas_call(
        body,
        out_shape=jax.ShapeDtypeStruct((N, C, (H - 4) * (W - 4)), jnp.float32),
        grid=(N,),
        in_specs=[
            pl.BlockSpec((None, C, H * W), lambda b: (b, 0, 0)),
            pl.BlockSpec((3 * C, 3 * C), lambda b: (0, 0)),
            pl.BlockSpec((C, C), lambda b: (0, 0)),
            pl.BlockSpec((3 * C, 3 * C), lambda b: (0, 0)),
            pl.BlockSpec((C, C), lambda b: (0, 0)),
        ],
        out_specs=pl.BlockSpec((None, C, (H - 4) * (W - 4)),
                               lambda b: (b, 0, 0)),
        compiler_params=pltpu.CompilerParams(
            dimension_semantics=("parallel",),
            vmem_limit_bytes=64 * 1024 * 1024),
    )(xf, w1c, b1k, w2c, b2k)
    return out.reshape(N, C, H - 4, W - 4)
